# baseline (speedup 1.0000x reference)
import jax
import jax.numpy as jnp
from jax.experimental import pallas as pl
from jax.experimental.pallas import tpu as pltpu

_NCHUNK = 8


def _dma_copy(h_ref, o_ref, sems):
    C = 1000 // _NCHUNK
    for i in range(_NCHUNK):
        pltpu.make_async_copy(h_ref.at[pl.ds(i * C, C)],
                              o_ref.at[pl.ds(i * C, C)], sems.at[i]).start()
    for i in range(_NCHUNK):
        pltpu.make_async_copy(h_ref.at[pl.ds(i * C, C)],
                              o_ref.at[pl.ds(i * C, C)], sems.at[i]).wait()


def kernel(encoded_sents, indices, hiddens, keys, U, V, W):
    B, N, D = hiddens.shape
    ht = jnp.transpose(hiddens, (1, 2, 0))
    out_t = pl.pallas_call(
        _dma_copy,
        in_specs=[pl.BlockSpec(memory_space=pl.ANY)],
        out_specs=pl.BlockSpec(memory_space=pl.ANY),
        out_shape=jax.ShapeDtypeStruct((N, D, B), jnp.float32),
        scratch_shapes=[pltpu.SemaphoreType.DMA((_NCHUNK,))],
    )(ht)
    return jnp.transpose(out_t, (2, 0, 1))


# speedup vs baseline: 48.5228x; 48.5228x over previous
import functools

import jax
import jax.numpy as jnp
from jax.experimental import pallas as pl
from jax.experimental.pallas import tpu as pltpu

_NC = 100


def _stream_kernel(T, NC, idx_ref, h_ref, s_ref, u_ref, v_ref, w_ref,
                   *k_refs_and_out):
    k_refs = k_refs_and_out[:T]
    out_ref = k_refs_and_out[T]
    out_ref[...] = h_ref[...]
    n0 = pl.program_id(0) * NC
    s = s_ref[...]
    for t in range(T):
        row = idx_ref[t] - n0

        @pl.when((row >= 0) & (row < NC))
        def _():
            h_i = out_ref[pl.ds(row, 1)][0]
            k_i = k_refs[t][0]
            g = jax.nn.sigmoid(jnp.sum(s * (h_i + k_i), axis=0,
                                       keepdims=True))
            h_tilde = jnp.maximum(
                jnp.dot(u_ref[...], h_i, preferred_element_type=jnp.float32)
                + jnp.dot(v_ref[...], k_i, preferred_element_type=jnp.float32)
                + jnp.dot(w_ref[...], s, preferred_element_type=jnp.float32),
                0.0,
            )
            h_new = h_i + g * h_tilde
            norm = jnp.sqrt(jnp.maximum(
                jnp.sum(h_new * h_new, axis=0, keepdims=True), 1e-12))
            out_ref[pl.ds(row, 1)] = (h_new / norm)[None]


@jax.jit
def kernel(encoded_sents, indices, hiddens, keys, U, V, W):
    B, N, D = hiddens.shape
    T = indices.shape[0]
    indices = indices.astype(jnp.int32)

    ht = jnp.transpose(hiddens, (1, 2, 0))
    kt = jnp.transpose(keys, (1, 2, 0))
    st = encoded_sents.T

    def k_spec(t):
        return pl.BlockSpec((1, D, B), lambda n, idx, _t=t: (idx[_t], 0, 0))

    out_t = pl.pallas_call(
        functools.partial(_stream_kernel, T, _NC),
        grid_spec=pltpu.PrefetchScalarGridSpec(
            num_scalar_prefetch=1,
            grid=(N // _NC,),
            in_specs=[
                pl.BlockSpec((_NC, D, B), lambda n, idx: (n, 0, 0)),
                pl.BlockSpec((D, B), lambda n, idx: (0, 0)),
                pl.BlockSpec((D, D), lambda n, idx: (0, 0)),
                pl.BlockSpec((D, D), lambda n, idx: (0, 0)),
                pl.BlockSpec((D, D), lambda n, idx: (0, 0)),
            ] + [k_spec(t) for t in range(T)],
            out_specs=pl.BlockSpec((_NC, D, B), lambda n, idx: (n, 0, 0)),
        ),
        out_shape=jax.ShapeDtypeStruct((N, D, B), jnp.float32),
        compiler_params=pltpu.CompilerParams(
            dimension_semantics=("parallel",)),
    )(indices, ht, st, U, V, W, *([kt] * T))
    return jnp.transpose(out_t, (2, 0, 1))


# speedup vs baseline: 48.5488x; 1.0005x over previous
import functools

import jax
import jax.numpy as jnp
from jax.experimental import pallas as pl
from jax.experimental.pallas import tpu as pltpu

_NC = 100


def _stream_kernel(T, NC, idx_ref, h_ref, s_ref, u_ref, v_ref, w_ref,
                   *k_refs_and_out):
    k_refs = k_refs_and_out[:T]
    out_ref = k_refs_and_out[T]
    out_ref[...] = h_ref[...]
    n0 = pl.program_id(0) * NC
    s = s_ref[...]
    for t in range(T):
        row = idx_ref[t] - n0

        @pl.when((row >= 0) & (row < NC))
        def _():
            h_i = out_ref[pl.ds(row, 1)][0]
            k_i = k_refs[t][0]
            g = jax.nn.sigmoid(jnp.sum(s * (h_i + k_i), axis=0,
                                       keepdims=True))
            h_tilde = jnp.maximum(
                jnp.dot(u_ref[...], h_i, preferred_element_type=jnp.float32)
                + jnp.dot(v_ref[...], k_i, preferred_element_type=jnp.float32)
                + jnp.dot(w_ref[...], s, preferred_element_type=jnp.float32),
                0.0,
            )
            h_new = h_i + g * h_tilde
            norm = jnp.sqrt(jnp.maximum(
                jnp.sum(h_new * h_new, axis=0, keepdims=True), 1e-12))
            out_ref[pl.ds(row, 1)] = (h_new / norm)[None]


@jax.jit
def kernel(encoded_sents, indices, hiddens, keys, U, V, W):
    B, N, D = hiddens.shape
    T = indices.shape[0]
    indices = indices.astype(jnp.int32)

    ht = jnp.transpose(hiddens, (1, 2, 0))
    kt = jnp.transpose(keys, (1, 2, 0))
    st = encoded_sents.T

    def k_spec(t):
        return pl.BlockSpec((1, D, B), lambda n, idx, _t=t: (idx[_t], 0, 0))

    out_t = pl.pallas_call(
        functools.partial(_stream_kernel, T, _NC),
        grid_spec=pltpu.PrefetchScalarGridSpec(
            num_scalar_prefetch=1,
            grid=(N // _NC,),
            in_specs=[
                pl.BlockSpec((_NC, D, B), lambda n, idx: (n, 0, 0)),
                pl.BlockSpec((D, B), lambda n, idx: (0, 0)),
                pl.BlockSpec((D, D), lambda n, idx: (0, 0)),
                pl.BlockSpec((D, D), lambda n, idx: (0, 0)),
                pl.BlockSpec((D, D), lambda n, idx: (0, 0)),
            ] + [k_spec(t) for t in range(T)],
            out_specs=pl.BlockSpec((_NC, D, B), lambda n, idx: (n, 0, 0)),
        ),
        out_shape=jax.ShapeDtypeStruct((N, D, B), jnp.float32),
    )(indices, ht, st, U, V, W, *([kt] * T))
    return jnp.transpose(out_t, (2, 0, 1))
